# baseline (speedup 1.0000x reference)
import functools

import jax
import jax.numpy as jnp
from jax import lax
from jax.experimental import pallas as pl
from jax.experimental.pallas import tpu as pltpu
from jax.experimental.pallas import tpu_sc as plsc

_B, _T, _D = 4, 2048, 1024
_B_TC = 3
_NC, _NS = 2, 16
_NW = _NC * _NS
_TPW = _T // _NW
_R = 16
_NITEMS = _TPW // _R
_UNROLL = 8
_CPR = _D // 16


def _sc_body(feat_hbm, sin_hbm, out_hbm, sin_v, fb0, fb1, si0, si1, so0, so1):
    wid = lax.axis_index("s") * _NC + lax.axis_index("c")
    t0 = wid * _TPW
    bsc = _B_TC

    pltpu.async_copy(feat_hbm.at[bsc, pl.ds(t0, _R)], fb0, si0)

    def stage(i, row, cur, si_cur, so_cur, nxt, si_nxt, so_nxt):
        @pl.when(i > 0)
        def _():
            pltpu.make_async_copy(nxt, out_hbm.at[0, pl.ds(0, _R)], so_nxt).wait()

        @pl.when(i < _NITEMS - 1)
        def _():
            pltpu.async_copy(feat_hbm.at[bsc, pl.ds(row + _R, _R)], nxt, si_nxt)

        pltpu.make_async_copy(feat_hbm.at[bsc, pl.ds(0, _R)], cur, si_cur).wait()

        for r in range(_R):
            @plsc.parallel_loop(0, _CPR, 1, unroll=_UNROLL)
            def add_col(c, r=r):
                sl = pl.ds(c * 16, 16)
                plsc.addupdate(cur.at[r, sl], sin_v[r, sl])

        pltpu.async_copy(cur, out_hbm.at[0, pl.ds(row, _R)], so_cur)

    def per_item(i, carry):
        row = t0 + i * _R
        pltpu.sync_copy(sin_hbm.at[pl.ds(row, _R)], sin_v)
        p = lax.bitwise_and(i, 1)

        @pl.when(p == 0)
        def _():
            stage(i, row, fb0, si0, so0, fb1, si1, so1)

        @pl.when(p == 1)
        def _():
            stage(i, row, fb1, si1, so1, fb0, si0, so0)

        return carry

    lax.fori_loop(0, _NITEMS, per_item, 0)

    if (_NITEMS - 1) % 2 == 0:
        pltpu.make_async_copy(fb0, out_hbm.at[0, pl.ds(0, _R)], so0).wait()
    else:
        pltpu.make_async_copy(fb1, out_hbm.at[0, pl.ds(0, _R)], so1).wait()


_sc_kernel = functools.partial(
    pl.kernel,
    out_type=jax.ShapeDtypeStruct((1, _T, _D), jnp.float32),
    mesh=plsc.VectorSubcoreMesh(core_axis_name="c", subcore_axis_name="s"),
    scratch_types=[
        pltpu.VMEM((_R, _D), jnp.float32),
        pltpu.VMEM((_R, _D), jnp.float32),
        pltpu.VMEM((_R, _D), jnp.float32),
        pltpu.SemaphoreType.DMA,
        pltpu.SemaphoreType.DMA,
        pltpu.SemaphoreType.DMA,
        pltpu.SemaphoreType.DMA,
    ],
)(_sc_body)


def _tc_add_kernel(f_ref, s_ref, o_ref):
    o_ref[...] = f_ref[...] + s_ref[...][None]


_BT_TC = 256


def kernel(features, sinusoids):
    B, T, D = features.shape
    sc_part = _sc_kernel(features, sinusoids)
    tc_part = pl.pallas_call(
        _tc_add_kernel,
        grid=(_B_TC, T // _BT_TC),
        in_specs=[
            pl.BlockSpec((1, _BT_TC, D), lambda b, j: (b, j, 0)),
            pl.BlockSpec((_BT_TC, D), lambda b, j: (j, 0)),
        ],
        out_specs=pl.BlockSpec((1, _BT_TC, D), lambda b, j: (b, j, 0)),
        out_shape=jax.ShapeDtypeStruct((_B_TC, T, D), features.dtype),
    )(features, sinusoids)
    return jnp.concatenate([tc_part, sc_part], axis=0)


# speedup vs baseline: 2.0365x; 2.0365x over previous
import jax
import jax.numpy as jnp
from jax.experimental import pallas as pl


def _add_kernel(f_ref, s_ref, o_ref):
    o_ref[...] = f_ref[...] + s_ref[...][None]


def kernel(features, sinusoids):
    B, T, D = features.shape
    BT = 256
    return pl.pallas_call(
        _add_kernel,
        grid=(T // BT, B),
        in_specs=[
            pl.BlockSpec((1, BT, D), lambda j, b: (b, j, 0)),
            pl.BlockSpec((BT, D), lambda j, b: (j, 0)),
        ],
        out_specs=pl.BlockSpec((1, BT, D), lambda j, b: (b, j, 0)),
        out_shape=jax.ShapeDtypeStruct((B, T, D), features.dtype),
    )(features, sinusoids)
